# baseline (speedup 1.0000x reference)
import functools

import jax
import jax.numpy as jnp
from jax import lax
from jax.experimental import pallas as pl
from jax.experimental.pallas import tpu as pltpu
from jax.experimental.pallas import tpu_sc as plsc

_B = 16384
_W = 1000
_KEEP = 100
_KEEP_PAD = 104
_NC = 2
_NS = 16
_NW = _NC * _NS
_RPW = _B // _NW
_C = 32
_NCHUNK = _RPW // _C
_NBUF = 2


def _sc_mask(x_hbm, out_hbm, buf, in_sem, out_sem):
    wid = lax.axis_index("s") * _NC + lax.axis_index("c")
    base = wid * _RPW

    zero = jnp.zeros((16,), jnp.float32)

    def _zero_row(r, carry):
        for off in list(range(0, _W - 16, 16)) + [_W - 16]:
            buf[r, pl.ds(off, 16)] = zero
        return carry

    lax.fori_loop(0, _NBUF * _C, _zero_row, 0)

    def _start_in(i, slot):
        row0 = base + i * _C
        return pltpu.async_copy(
            x_hbm.at[pl.ds(row0, _C), pl.ds(0, _KEEP_PAD)],
            buf.at[pl.ds(slot * _C, _C), pl.ds(0, _KEEP_PAD)],
            in_sem,
        )

    lane = lax.iota(jnp.int32, 16)

    def _fix_rows(slot):
        def _fix_row(r, carry):
            v = buf[slot * _C + r, pl.ds(96, 16)]
            buf[slot * _C + r, pl.ds(96, 16)] = jnp.where(
                lane < _KEEP - 96, v, 0.0)
            return carry

        lax.fori_loop(0, _C, _fix_row, 0)

    def _start_out(i, slot):
        row0 = base + i * _C
        return pltpu.async_copy(
            buf.at[pl.ds(slot * _C, _C)],
            out_hbm.at[pl.ds(row0, _C)],
            out_sem,
        )

    out_handles = [None] * _NCHUNK
    for i in range(_NCHUNK):
        slot = i % _NBUF
        if i >= _NBUF:
            out_handles[i - _NBUF].wait()
        _start_in(i, slot).wait()
        _fix_rows(slot)
        out_handles[i] = _start_out(i, slot)
    for i in range(_NCHUNK - _NBUF, _NCHUNK):
        out_handles[i].wait()


@functools.cache
def _build_sc_mask():
    mesh = plsc.VectorSubcoreMesh(core_axis_name="c", subcore_axis_name="s")
    return pl.kernel(
        _sc_mask,
        mesh=mesh,
        out_type=jax.ShapeDtypeStruct((_B, _W), jnp.float32),
        scratch_types=[
            pltpu.VMEM((_NBUF * _C, _W), jnp.float32),
            pltpu.SemaphoreType.DMA,
            pltpu.SemaphoreType.DMA,
        ],
        compiler_params=pltpu.CompilerParams(use_tc_tiling_on_sc=False),
    )


def kernel(sender_input, labels):
    del labels
    return _build_sc_mask()(sender_input)


# speedup vs baseline: 1.8122x; 1.8122x over previous
import functools

import jax
import jax.numpy as jnp
from jax import lax
from jax.experimental import pallas as pl
from jax.experimental.pallas import tpu as pltpu
from jax.experimental.pallas import tpu_sc as plsc

_B = 16384
_W = 1000
_KEEP = 100
_KEEP_PAD = 128
_NC = 2
_NS = 16
_NW = _NC * _NS
_RPW = _B // _NW
_C = 32
_NCHUNK = _RPW // _C
_NBUF = 2


def _sc_mask(x_hbm, out_hbm, buf, in_sem, out_sem):
    wid = lax.axis_index("s") * _NC + lax.axis_index("c")
    base = wid * _RPW

    zero = jnp.zeros((16,), jnp.float32)

    def _zero_row(r, carry):
        for off in list(range(0, _W - 16, 16)) + [_W - 16]:
            buf[r, pl.ds(off, 16)] = zero
        return carry

    lax.fori_loop(0, _NBUF * _C, _zero_row, 0)

    def _start_in(i, slot):
        row0 = base + i * _C
        return pltpu.async_copy(
            x_hbm.at[pl.ds(row0, _C), pl.ds(0, _KEEP_PAD)],
            buf.at[pl.ds(slot * _C, _C), pl.ds(0, _KEEP_PAD)],
            in_sem,
        )

    lane = lax.iota(jnp.int32, 16)

    zero16 = jnp.zeros((16,), jnp.float32)

    def _fix_rows(slot):
        def _fix_row(r, carry):
            v = buf[slot * _C + r, pl.ds(96, 16)]
            buf[slot * _C + r, pl.ds(96, 16)] = jnp.where(
                lane < _KEEP - 96, v, 0.0)
            buf[slot * _C + r, pl.ds(112, 16)] = zero16
            return carry

        lax.fori_loop(0, _C, _fix_row, 0)

    def _start_out(i, slot):
        row0 = base + i * _C
        return pltpu.async_copy(
            buf.at[pl.ds(slot * _C, _C)],
            out_hbm.at[pl.ds(row0, _C)],
            out_sem,
        )

    out_handles = [None] * _NCHUNK
    for i in range(_NCHUNK):
        slot = i % _NBUF
        if i >= _NBUF:
            out_handles[i - _NBUF].wait()
        _start_in(i, slot).wait()
        _fix_rows(slot)
        out_handles[i] = _start_out(i, slot)
    for i in range(_NCHUNK - _NBUF, _NCHUNK):
        out_handles[i].wait()


@functools.cache
def _build_sc_mask():
    mesh = plsc.VectorSubcoreMesh(core_axis_name="c", subcore_axis_name="s")
    return pl.kernel(
        _sc_mask,
        mesh=mesh,
        out_type=jax.ShapeDtypeStruct((_B, _W), jnp.float32),
        scratch_types=[
            pltpu.VMEM((_NBUF * _C, _W), jnp.float32),
            pltpu.SemaphoreType.DMA,
            pltpu.SemaphoreType.DMA,
        ],
    )


def kernel(sender_input, labels):
    del labels
    return _build_sc_mask()(sender_input)


# speedup vs baseline: 1.8664x; 1.0299x over previous
import jax
import jax.numpy as jnp
from jax.experimental import pallas as pl
from jax.experimental.pallas import tpu as pltpu

_B = 16384
_W = 1000
_KEEP = 100
_RB = 512
_NCHUNK = _B // _RB
_NB = 4


def _body(x_hbm, o_hbm, buf, in_sems, out_sems):
    zero = jnp.zeros((_RB, _W), jnp.float32)
    for s in range(_NB):
        buf[s] = zero

    lane = jax.lax.broadcasted_iota(jnp.int32, (_RB, 128), 1)

    def in_copy(i, slot):
        return pltpu.make_async_copy(
            x_hbm.at[pl.ds(i * _RB, _RB), pl.ds(0, 128)],
            buf.at[slot, :, pl.ds(0, 128)],
            in_sems.at[slot],
        )

    def out_copy(i, slot):
        return pltpu.make_async_copy(
            buf.at[slot],
            o_hbm.at[pl.ds(i * _RB, _RB)],
            out_sems.at[slot],
        )

    for i in range(_NCHUNK):
        slot = i % _NB
        if i >= _NB:
            out_copy(i - _NB, slot).wait()
        in_copy(i, slot).start()
        in_copy(i, slot).wait()
        buf[slot, :, :128] = jnp.where(lane < _KEEP, buf[slot, :, :128], 0.0)
        out_copy(i, slot).start()
    for i in range(_NCHUNK - _NB, _NCHUNK):
        out_copy(i, i % _NB).wait()


def kernel(sender_input, labels):
    del labels
    return pl.pallas_call(
        _body,
        in_specs=[pl.BlockSpec(memory_space=pl.ANY)],
        out_specs=pl.BlockSpec(memory_space=pl.ANY),
        out_shape=jax.ShapeDtypeStruct((_B, _W), jnp.float32),
        scratch_shapes=[
            pltpu.VMEM((_NB, _RB, _W), jnp.float32),
            pltpu.SemaphoreType.DMA((_NB,)),
            pltpu.SemaphoreType.DMA((_NB,)),
        ],
    )(sender_input)


# speedup vs baseline: 11.5957x; 6.2128x over previous
import jax
import jax.numpy as jnp
from jax.experimental import pallas as pl

_B = 16384
_W = 1000
_KEEP = 100
_RPAD = 104
_CB = 1024


def _body(x_ref, o_ref):
    row = jax.lax.broadcasted_iota(jnp.int32, (_RPAD, _CB), 0)
    o_ref[:_RPAD, :] = jnp.where(row < _KEEP, x_ref[...], 0.0)
    o_ref[_RPAD:, :] = jnp.zeros((_W - _RPAD, _CB), jnp.float32)


def kernel(sender_input, labels):
    del labels
    xt = sender_input.T
    yt = pl.pallas_call(
        _body,
        grid=(_B // _CB,),
        in_specs=[pl.BlockSpec((_RPAD, _CB), lambda j: (0, j))],
        out_specs=pl.BlockSpec((_W, _CB), lambda j: (0, j)),
        out_shape=jax.ShapeDtypeStruct((_W, _B), jnp.float32),
    )(xt)
    return yt.T
